# Initial kernel scaffold; baseline (speedup 1.0000x reference)
import jax
import jax.numpy as jnp
from jax.experimental import pallas as pl


def kernel(x, token_table, position_table):
    raise NotImplementedError("write your pallas kernel here")


# speedup vs baseline: 1.2885x; 1.2885x over previous
import jax
import jax.numpy as jnp
from jax import lax
from jax.experimental import pallas as pl
from jax.experimental.pallas import tpu as pltpu
from jax.experimental.pallas import tpu_sc as plsc

VOCAB_SIZE = 1_000_000
N_EMBD = 64
SEQ_LEN = 200
BATCH = 1024

_info = plsc.get_sparse_core_info()
_NC, _NS = _info.num_cores, _info.num_subcores
NW = _NC * _NS
SEQ_PER_W = BATCH // NW
HALF = SEQ_LEN // 2


def _emb_body(x_hbm, tok_hbm, pos_hbm, out_hbm, idx_v, pos_v, rows_v, gsem):
    cid = lax.axis_index("c")
    sid = lax.axis_index("s")
    wid = sid * _NC + cid

    pltpu.sync_copy(pos_hbm, pos_v)
    pltpu.sync_copy(x_hbm.at[pl.ds(wid * SEQ_PER_W, SEQ_PER_W)], idx_v)

    def seq_body(s, carry):
        cp0 = pltpu.async_copy(
            tok_hbm.at[idx_v.at[s, 0]], rows_v.at[pl.ds(0, HALF)], gsem)
        cp1 = pltpu.async_copy(
            tok_hbm.at[idx_v.at[s, 1]], rows_v.at[pl.ds(HALF, HALF)], gsem)
        cp0.wait()
        cp1.wait()

        def row_body(r, c):
            for k in range(4):
                rr = r * 4 + k
                for b in range(N_EMBD // 16):
                    sl = pl.ds(b * 16, 16)
                    rows_v[rr, sl] = rows_v[rr, sl] + pos_v[rr, sl]
            return c

        lax.fori_loop(0, SEQ_LEN // 4, row_body, 0)

        out_base = (wid * SEQ_PER_W + s) * SEQ_LEN
        pltpu.sync_copy(rows_v, out_hbm.at[pl.ds(out_base, SEQ_LEN)])
        return carry

    lax.fori_loop(0, SEQ_PER_W, seq_body, 0)


def kernel(x, token_table, position_table):
    x3 = x.reshape(BATCH, 2, HALF).astype(jnp.int32)
    run = pl.kernel(
        _emb_body,
        out_type=jax.ShapeDtypeStruct((BATCH * SEQ_LEN, N_EMBD), jnp.float32),
        mesh=plsc.VectorSubcoreMesh(core_axis_name="c", subcore_axis_name="s"),
        scratch_types=[
            pltpu.VMEM((SEQ_PER_W, 2, HALF), jnp.int32),
            pltpu.VMEM((SEQ_LEN, N_EMBD), jnp.float32),
            pltpu.VMEM((SEQ_LEN, N_EMBD), jnp.float32),
            pltpu.SemaphoreType.DMA,
        ],
        compiler_params=pltpu.CompilerParams(use_tc_tiling_on_sc=False),
    )
    out = run(x3, token_table, position_table)
    return out.reshape(BATCH, SEQ_LEN, N_EMBD)


# speedup vs baseline: 1.3730x; 1.0656x over previous
import jax
import jax.numpy as jnp
from jax import lax
from jax.experimental import pallas as pl
from jax.experimental.pallas import tpu as pltpu
from jax.experimental.pallas import tpu_sc as plsc

VOCAB_SIZE = 1_000_000
N_EMBD = 64
SEQ_LEN = 200
BATCH = 1024

_info = plsc.get_sparse_core_info()
_NC, _NS = _info.num_cores, _info.num_subcores
NW = _NC * _NS
SEQ_PER_W = BATCH // NW
HALF = SEQ_LEN // 2
NB = 4


def _emb_body(x_hbm, tok_hbm, pos_hbm, out_hbm, idx_v, pos_v, gbuf, sbuf,
              gsems, ssems):
    cid = lax.axis_index("c")
    sid = lax.axis_index("s")
    wid = sid * _NC + cid

    pltpu.sync_copy(pos_hbm, pos_v)
    pltpu.sync_copy(x_hbm.at[pl.ds(wid * SEQ_PER_W, SEQ_PER_W)], idx_v)

    def gather(s, b):
        pltpu.async_copy(tok_hbm.at[idx_v.at[s, 0]],
                         gbuf.at[b, pl.ds(0, HALF)], gsems.at[b])
        pltpu.async_copy(tok_hbm.at[idx_v.at[s, 1]],
                         gbuf.at[b, pl.ds(HALF, HALF)], gsems.at[b])

    def wait_gather(b):
        pltpu.make_async_copy(tok_hbm.at[idx_v.at[0, 0]],
                              gbuf.at[b, pl.ds(0, HALF)], gsems.at[b]).wait()
        pltpu.make_async_copy(tok_hbm.at[idx_v.at[0, 1]],
                              gbuf.at[b, pl.ds(HALF, HALF)], gsems.at[b]).wait()

    def store(s, b):
        out_base = (wid * SEQ_PER_W + s) * SEQ_LEN
        pltpu.async_copy(sbuf.at[b], out_hbm.at[pl.ds(out_base, SEQ_LEN)],
                         ssems.at[b])

    def wait_store(b):
        pltpu.make_async_copy(sbuf.at[b], out_hbm.at[pl.ds(0, SEQ_LEN)],
                              ssems.at[b]).wait()

    for b in range(NB):
        gather(b, b)

    def stage(i, carry):
        for b in range(NB):
            s = i * NB + b
            wait_gather(b)

            @pl.when(s >= NB)
            def _():
                wait_store(b)

            def add_body(r, c):
                base = r * 8
                for k in range(8):
                    rr = base + k
                    for blk in range(N_EMBD // 16):
                        sl = pl.ds(blk * 16, 16)
                        sbuf[b, rr, sl] = gbuf[b, rr, sl] + pos_v[rr, sl]
                return c

            lax.fori_loop(0, SEQ_LEN // 8, add_body, 0)
            store(s, b)

            @pl.when(s + NB < SEQ_PER_W)
            def _():
                gather(s + NB, b)
        return carry

    lax.fori_loop(0, SEQ_PER_W // NB, stage, 0)

    for b in range(NB):
        wait_store(b)


def kernel(x, token_table, position_table):
    x3 = x.reshape(BATCH, 2, HALF).astype(jnp.int32)
    run = pl.kernel(
        _emb_body,
        out_type=jax.ShapeDtypeStruct((BATCH * SEQ_LEN, N_EMBD), jnp.float32),
        mesh=plsc.VectorSubcoreMesh(core_axis_name="c", subcore_axis_name="s"),
        scratch_types=[
            pltpu.VMEM((SEQ_PER_W, 2, HALF), jnp.int32),
            pltpu.VMEM((SEQ_LEN, N_EMBD), jnp.float32),
            pltpu.VMEM((NB, SEQ_LEN, N_EMBD), jnp.float32),
            pltpu.VMEM((NB, SEQ_LEN, N_EMBD), jnp.float32),
            pltpu.SemaphoreType.DMA((NB,)),
            pltpu.SemaphoreType.DMA((NB,)),
        ],
        compiler_params=pltpu.CompilerParams(use_tc_tiling_on_sc=False),
    )
    out = run(x3, token_table, position_table)
    return out.reshape(BATCH, SEQ_LEN, N_EMBD)
